# baseline (speedup 1.0000x reference)
import jax
import jax.numpy as jnp
from jax.experimental import pallas as pl
from jax.experimental.pallas import tpu as pltpu

_SCORE_THR = 0.05
_IOU_THR = 0.5
_N = 5000
_NPAD = 5120
_BT = 128
_NB = _NPAD // _BT


def _nms_kernel(band_lo_ref, band_hi_ref, data_c_ref, data_r_ref, out_ref,
                keep_ref, acc_ref):
    n = _NPAD

    scores_row = data_r_ref[4:5, :]
    valid = (scores_row >= _SCORE_THR).astype(jnp.float32)
    keep_ref[0:1, :] = valid

    max_coord = jnp.max(data_r_ref[0:4, :])
    off_scale = max_coord + 1.0

    def sweep(state):
        _, t = state

        def ib_body(ib, carry):
            i0 = ib * _BT
            offi = data_r_ref[5:6, pl.ds(i0, _BT)] * off_scale
            xi1 = data_r_ref[0:1, pl.ds(i0, _BT)] + offi
            yi1 = data_r_ref[1:2, pl.ds(i0, _BT)] + offi
            xi2 = data_r_ref[2:3, pl.ds(i0, _BT)] + offi
            yi2 = data_r_ref[3:4, pl.ds(i0, _BT)] + offi
            si = data_r_ref[4:5, pl.ds(i0, _BT)]
            ii = data_r_ref[6:7, pl.ds(i0, _BT)]
            ai = (xi2 - xi1 + 1.0) * (yi2 - yi1 + 1.0)

            def jb_body(jb, acc):
                j0 = jb * _BT
                cj_all = data_c_ref[pl.ds(j0, _BT), :]
                offj = cj_all[:, 5:6] * off_scale
                xj1 = cj_all[:, 0:1] + offj
                yj1 = cj_all[:, 1:2] + offj
                xj2 = cj_all[:, 2:3] + offj
                yj2 = cj_all[:, 3:4] + offj
                sj = cj_all[:, 4:5]
                jj = cj_all[:, 6:7]
                aj = (xj2 - xj1 + 1.0) * (yj2 - yj1 + 1.0)

                xmin = jnp.maximum(xj1, xi1)
                ymin = jnp.maximum(yj1, yi1)
                xmax = jnp.minimum(xj2, xi2)
                ymax = jnp.minimum(yj2, yi2)
                inter = (jnp.maximum(xmax - xmin, 0.0)
                         * jnp.maximum(ymax - ymin, 0.0))
                iou = inter / (aj + ai - inter)
                prec = (sj > si) | ((sj == si) & (jj < ii))
                sf = ((iou > _IOU_THR) & prec).astype(jnp.float32)

                kj = keep_ref[0:1, pl.ds(j0, _BT)]
                kj8 = jnp.broadcast_to(kj, (8, _BT))
                contrib = jax.lax.dot(kj8, sf,
                                      preferred_element_type=jnp.float32)
                return acc + contrib[0:1, :]

            acc = jax.lax.fori_loop(
                band_lo_ref[ib], band_hi_ref[ib], jb_body,
                jnp.zeros((1, _BT), jnp.float32))
            acc_ref[0:1, pl.ds(i0, _BT)] = acc
            return carry

        jax.lax.fori_loop(0, _NB, ib_body, 0)

        old = keep_ref[0:1, :]
        new = valid * (acc_ref[0:1, :] < 0.5).astype(jnp.float32)
        keep_ref[0:1, :] = new
        changed = jnp.max(jnp.abs(new - old)) > 0.0
        return changed, t + 1

    jax.lax.while_loop(lambda s: s[0] & (s[1] < n + 2), sweep,
                       (True, jnp.int32(0)))

    k = keep_ref[0:1, :]
    out_ref[0:4, :] = data_r_ref[0:4, :] * k
    out_ref[4:5, :] = data_r_ref[4:5, :] * k


def _nms_call(band_lo, band_hi, data_c, data_r, interpret=False):
    return pl.pallas_call(
        _nms_kernel,
        out_shape=jax.ShapeDtypeStruct((5, _NPAD), jnp.float32),
        in_specs=[
            pl.BlockSpec(memory_space=pltpu.SMEM),
            pl.BlockSpec(memory_space=pltpu.SMEM),
            pl.BlockSpec(),
            pl.BlockSpec(),
        ],
        scratch_shapes=[
            pltpu.VMEM((8, _NPAD), jnp.float32),
            pltpu.VMEM((8, _NPAD), jnp.float32),
        ],
        interpret=interpret,
    )(band_lo, band_hi, data_c, data_r)


def _prep(boxes, scores, class_ids):
    perm = jnp.argsort(class_ids, stable=True)
    b = boxes[perm]
    s = scores[perm]
    c = class_ids[perm].astype(jnp.float32)
    idxf = perm.astype(jnp.float32)

    npad = _NPAD - _N
    b = jnp.pad(b, ((0, npad), (0, 0)))
    s = jnp.pad(s, (0, npad), constant_values=-1.0)
    c = jnp.pad(c, (0, npad), constant_values=81.0)
    idxf = jnp.pad(idxf, (0, npad), constant_values=float(_NPAD))
    data_c = jnp.concatenate(
        [b, s[:, None], c[:, None], idxf[:, None]], axis=1)
    data_r = data_c.T

    ci = c.astype(jnp.int32).reshape(_NB, _BT)
    tmin = ci.min(axis=1)
    tmax = ci.max(axis=1)
    band_lo = jnp.sum(tmax[None, :] < tmin[:, None], axis=1,
                      dtype=jnp.int32)
    band_hi = _NB - jnp.sum(tmin[None, :] > tmax[:, None], axis=1,
                            dtype=jnp.int32)
    return band_lo, band_hi, data_c, data_r, perm


def kernel(boxes, scores, class_ids):
    band_lo, band_hi, data_c, data_r, perm = _prep(boxes, scores, class_ids)
    out = _nms_call(band_lo, band_hi, data_c, data_r)
    outp = out.T[:_N]
    return jnp.zeros((_N, 5), jnp.float32).at[perm].set(outp)


# speedup vs baseline: 1.7143x; 1.7143x over previous
import jax
import jax.numpy as jnp
from jax.experimental import pallas as pl
from jax.experimental.pallas import tpu as pltpu

_SCORE_THR = 0.05
_IOU_THR = 0.5
_N = 5000
_NPAD = 5120
_BT = 256
_NB = _NPAD // _BT


def _nms_kernel(band_lo_ref, band_hi_ref, data_c_ref, data_r_ref, out_ref,
                keep_ref, delta_ref, acc_ref, flag_ref):
    n = _NPAD

    scores_row = data_r_ref[4:5, :]
    valid = (scores_row >= _SCORE_THR).astype(jnp.float32)
    keep_ref[0:1, :] = valid
    delta_ref[0:1, :] = valid
    acc_ref[0:1, :] = jnp.zeros((1, n), jnp.float32)

    def init_flags(jb, c):
        flag_ref[jb] = 1.0
        return c

    jax.lax.fori_loop(0, _NB, init_flags, 0)

    max_coord = jnp.max(data_r_ref[0:4, :])
    off_scale = max_coord + 1.0

    def sweep(state):
        _, t = state

        def jb_body(jb, carry):
            @pl.when(flag_ref[jb] != 0.0)
            def _():
                j0 = jb * _BT
                cj_all = data_c_ref[pl.ds(j0, _BT), :]
                offj = cj_all[:, 5:6] * off_scale
                shape = (_BT, _BT)
                xj1 = jnp.broadcast_to(cj_all[:, 0:1] + offj, shape)
                yj1 = jnp.broadcast_to(cj_all[:, 1:2] + offj, shape)
                xj2 = jnp.broadcast_to(cj_all[:, 2:3] + offj, shape)
                yj2 = jnp.broadcast_to(cj_all[:, 3:4] + offj, shape)
                sj = jnp.broadcast_to(cj_all[:, 4:5], shape)
                jj = jnp.broadcast_to(cj_all[:, 6:7], shape)
                aj = (xj2 - xj1 + 1.0) * (yj2 - yj1 + 1.0)

                dj = delta_ref[0:1, pl.ds(j0, _BT)]
                dj8 = jnp.broadcast_to(dj, (8, _BT))

                def ib_body(ib, c):
                    i0 = ib * _BT
                    offi = data_r_ref[5:6, pl.ds(i0, _BT)] * off_scale
                    xi1 = data_r_ref[0:1, pl.ds(i0, _BT)] + offi
                    yi1 = data_r_ref[1:2, pl.ds(i0, _BT)] + offi
                    xi2 = data_r_ref[2:3, pl.ds(i0, _BT)] + offi
                    yi2 = data_r_ref[3:4, pl.ds(i0, _BT)] + offi
                    si = data_r_ref[4:5, pl.ds(i0, _BT)]
                    ii = data_r_ref[6:7, pl.ds(i0, _BT)]
                    ai = (xi2 - xi1 + 1.0) * (yi2 - yi1 + 1.0)

                    xmin = jnp.maximum(xj1, xi1)
                    ymin = jnp.maximum(yj1, yi1)
                    xmax = jnp.minimum(xj2, xi2)
                    ymax = jnp.minimum(yj2, yi2)
                    inter = (jnp.maximum(xmax - xmin, 0.0)
                             * jnp.maximum(ymax - ymin, 0.0))
                    iou = inter / (aj + ai - inter)
                    prec = (sj > si) | ((sj == si) & (jj < ii))
                    sf = ((iou > _IOU_THR) & prec).astype(jnp.float32)

                    contrib = jax.lax.dot(dj8, sf,
                                          preferred_element_type=jnp.float32)
                    acc_ref[0:1, pl.ds(i0, _BT)] += contrib[0:1, :]
                    return c

                jax.lax.fori_loop(band_lo_ref[jb], band_hi_ref[jb],
                                  ib_body, 0)

            return carry

        jax.lax.fori_loop(0, _NB, jb_body, 0)

        old = keep_ref[0:1, :]
        new = valid * (acc_ref[0:1, :] < 0.5).astype(jnp.float32)
        delta = new - old
        keep_ref[0:1, :] = new
        delta_ref[0:1, :] = delta
        def set_flags(jb, c):
            flag_ref[jb] = jnp.max(jnp.abs(delta_ref[0:1, pl.ds(jb * _BT, _BT)]))
            return c

        jax.lax.fori_loop(0, _NB, set_flags, 0)
        changed = jnp.max(jnp.abs(delta)) > 0.0
        return changed, t + 1

    jax.lax.while_loop(lambda s: s[0] & (s[1] < n + 2), sweep,
                       (True, jnp.int32(0)))

    k = keep_ref[0:1, :]
    out_ref[0:4, :] = data_r_ref[0:4, :] * k
    out_ref[4:5, :] = data_r_ref[4:5, :] * k


def _nms_call(band_lo, band_hi, data_c, data_r, interpret=False):
    return pl.pallas_call(
        _nms_kernel,
        out_shape=jax.ShapeDtypeStruct((5, _NPAD), jnp.float32),
        in_specs=[
            pl.BlockSpec(memory_space=pltpu.SMEM),
            pl.BlockSpec(memory_space=pltpu.SMEM),
            pl.BlockSpec(),
            pl.BlockSpec(),
        ],
        scratch_shapes=[
            pltpu.VMEM((8, _NPAD), jnp.float32),
            pltpu.VMEM((8, _NPAD), jnp.float32),
            pltpu.VMEM((8, _NPAD), jnp.float32),
            pltpu.SMEM((_NB,), jnp.float32),
        ],
        interpret=interpret,
    )(band_lo, band_hi, data_c, data_r)


def _prep(boxes, scores, class_ids):
    perm = jnp.argsort(class_ids, stable=True)
    idxf = jnp.arange(_N, dtype=jnp.float32)
    data = jnp.concatenate(
        [boxes, scores[:, None], class_ids.astype(jnp.float32)[:, None],
         idxf[:, None]], axis=1)
    datap = data[perm]

    npad = _NPAD - _N
    pad_row = jnp.array([[0.0, 0.0, 0.0, 0.0, -1.0, 81.0, float(_NPAD)]],
                        jnp.float32)
    data_c = jnp.concatenate(
        [datap, jnp.broadcast_to(pad_row, (npad, 7))], axis=0)
    data_r = data_c.T

    ci = data_c[:, 5].astype(jnp.int32).reshape(_NB, _BT)
    tmin = ci.min(axis=1)
    tmax = ci.max(axis=1)
    band_lo = jnp.sum(tmax[None, :] < tmin[:, None], axis=1,
                      dtype=jnp.int32)
    band_hi = _NB - jnp.sum(tmin[None, :] > tmax[:, None], axis=1,
                            dtype=jnp.int32)
    return band_lo, band_hi, data_c, data_r, perm


def kernel(boxes, scores, class_ids):
    band_lo, band_hi, data_c, data_r, perm = _prep(boxes, scores, class_ids)
    out = _nms_call(band_lo, band_hi, data_c, data_r)
    outp = out.T[:_N]
    return jnp.zeros((_N, 5), jnp.float32).at[perm].set(outp)
